# baseline (speedup 1.0000x reference)
import functools

import jax
import jax.numpy as jnp
from jax import lax
from jax.experimental import pallas as pl
from jax.experimental.pallas import tpu as pltpu

_R = 0.3
_TOP_K = 2


def _gate_router_block(x_ref, w_ref, b_ref, probs_ref, idx_ref,
                       sum_ref, max_ref, *, s_total, n_chunks):
    j = pl.program_id(1)
    xb = x_ref[...]
    psum = jnp.sum(xb, axis=1)
    pmax = jnp.max(xb, axis=1)

    @pl.when(j == 0)
    def _init():
        sum_ref[...] = psum
        max_ref[...] = pmax

    @pl.when(j > 0)
    def _acc():
        sum_ref[...] = sum_ref[...] + psum
        max_ref[...] = jnp.maximum(max_ref[...], pmax)

    @pl.when(j == n_chunks - 1)
    def _finish():
        avg = sum_ref[...] * (1.0 / s_total)
        feat = avg * (1.0 - _R) + max_ref[...] * _R
        scores = lax.dot_general(
            feat, w_ref[...],
            dimension_numbers=(((1,), (1,)), ((), ())),
            preferred_element_type=jnp.float32,
        ) + b_ref[...]

        e = scores.shape[1]
        iota = lax.broadcasted_iota(jnp.int32, scores.shape, 1)

        m1 = jnp.max(scores, axis=1, keepdims=True)
        idx1 = jnp.min(jnp.where(scores == m1, iota, e), axis=1, keepdims=True)
        masked = jnp.where(iota == idx1, -jnp.inf, scores)
        m2 = jnp.max(masked, axis=1, keepdims=True)
        idx2 = jnp.min(jnp.where(masked == m2, iota, e), axis=1, keepdims=True)

        e2 = jnp.exp(m2 - m1)
        denom = 1.0 + e2
        p1 = 1.0 / denom
        p2 = e2 / denom
        probs_ref[...] = (jnp.where(iota == idx1, p1, 0.0)
                          + jnp.where(iota == idx2, p2, 0.0))
        idx_ref[...] = jnp.concatenate([idx1, idx2], axis=1)


def kernel(x, W, b):
    B, D, H, Wsp = x.shape
    E = W.shape[0]
    S = H * Wsp
    xt = jnp.transpose(x, (0, 2, 3, 1)).reshape(B, S, D)
    b2 = b.reshape(1, E)

    BBLK = 16
    NCHUNK = 3
    SCHUNK = S // NCHUNK

    probs, indices = pl.pallas_call(
        functools.partial(_gate_router_block, s_total=S, n_chunks=NCHUNK),
        grid=(B // BBLK, NCHUNK),
        in_specs=[
            pl.BlockSpec((BBLK, SCHUNK, D), lambda i, j: (i, j, 0)),
            pl.BlockSpec((E, D), lambda i, j: (0, 0)),
            pl.BlockSpec((1, E), lambda i, j: (0, 0)),
        ],
        out_specs=[
            pl.BlockSpec((BBLK, E), lambda i, j: (i, 0)),
            pl.BlockSpec((BBLK, _TOP_K), lambda i, j: (i, 0)),
        ],
        out_shape=[
            jax.ShapeDtypeStruct((B, E), jnp.float32),
            jax.ShapeDtypeStruct((B, _TOP_K), jnp.int32),
        ],
        scratch_shapes=[
            pltpu.VMEM((BBLK, D), jnp.float32),
            pltpu.VMEM((BBLK, D), jnp.float32),
        ],
    )(xt, W, b2)
    return (probs, indices)
